# baseline (speedup 1.0000x reference)
import functools
import math

import jax
import jax.numpy as jnp
from jax import lax
from jax.experimental import pallas as pl
from jax.experimental.pallas import tpu as pltpu
from jax.experimental.pallas import tpu_sc as plsc

VOCAB = 100000
D = 128
BATCH = 4096
SEQ = 50
SEQ_PAD = 56
NW = 32
K_PARTS = 4
PART_B = BATCH // K_PARTS
B_PER_W = PART_B // NW
ROWS_PER_W = B_PER_W * SEQ_PAD
NB = 2
CHUNK = NB * SEQ_PAD
N_CHUNKS = B_PER_W // NB
NBUF = 4
LANES = 16
SCALE = math.sqrt(D)

_mesh = plsc.VectorSubcoreMesh(core_axis_name="c", subcore_axis_name="s")


@functools.partial(
    pl.kernel,
    out_type=jax.ShapeDtypeStruct((PART_B * SEQ_PAD, D), jnp.float32),
    mesh=_mesh,
    scratch_types=[
        pltpu.VMEM((ROWS_PER_W,), jnp.int32),
        pltpu.VMEM((NBUF, CHUNK, D), jnp.float32),
        pltpu.VMEM((NBUF, CHUNK, D), jnp.float32),
    ]
    + [pltpu.SemaphoreType.DMA] * (2 * NBUF),
)
def _gather_part(idx_hbm, table_hbm, out_hbm, idx_v, gbuf, sbuf, *sems):
    gsem = sems[:NBUF]
    ssem = sems[NBUF:]
    wid = lax.axis_index("s") * 2 + lax.axis_index("c")
    base = wid * ROWS_PER_W
    pltpu.sync_copy(idx_hbm.at[pl.ds(base, ROWS_PER_W)], idx_v)

    def gather_desc(c, b):
        return pltpu.make_async_copy(
            table_hbm.at[idx_v.at[pl.ds(c * CHUNK, CHUNK)]],
            gbuf.at[b], gsem[b])

    def store_desc(c, b):
        return pltpu.make_async_copy(
            sbuf.at[b], out_hbm.at[pl.ds(base + c * CHUNK, CHUNK)], ssem[b])

    for b in range(NBUF):
        gather_desc(b, b).start()

    for c in range(N_CHUNKS):
        b = c % NBUF
        gather_desc(c, b).wait()

        if c >= NBUF:
            store_desc(c - NBUF, b).wait()

        def scale_row(r, _):
            for j in range(D // LANES):
                sl = pl.ds(j * LANES, LANES)
                sbuf[b, r, sl] = gbuf[b, r, sl] * SCALE
            return 0

        lax.fori_loop(0, CHUNK, scale_row, 0)

        if c + NBUF < N_CHUNKS:
            gather_desc(c + NBUF, b).start()

        store_desc(c, b).start()

    for b in range(NBUF):
        store_desc(N_CHUNKS - NBUF + b, b).wait()


def kernel(input_ids, embed_weight):
    ids = input_ids.astype(jnp.int32)
    idx_pad = jnp.pad(ids, ((0, 0), (0, SEQ_PAD - SEQ))).reshape(
        BATCH * SEQ_PAD)
    f = jnp.zeros((BATCH, SEQ, D), jnp.float32)
    for k in range(K_PARTS):
        part_idx = lax.slice(
            idx_pad,
            (k * PART_B * SEQ_PAD,),
            ((k + 1) * PART_B * SEQ_PAD,),
        )
        raw2d = _gather_part(part_idx, embed_weight)
        part = raw2d.reshape(PART_B, SEQ_PAD, D)[:, :SEQ, :]
        f = lax.dynamic_update_slice(f, part, (k * PART_B, 0, 0))
    return f


# speedup vs baseline: 5.8251x; 5.8251x over previous
import functools
import math

import jax
import jax.numpy as jnp
from jax import lax
from jax.experimental import pallas as pl
from jax.experimental.pallas import tpu as pltpu
from jax.experimental.pallas import tpu_sc as plsc

VOCAB = 100000
D = 128
BATCH = 4096
SEQ = 50
SEQ_PAD = 56
NW = 32
K_PARTS = 4
PART_B = BATCH // K_PARTS
B_PER_W = PART_B // NW
ROWS_PER_W = B_PER_W * SEQ_PAD
NB = 2
CHUNK = NB * SEQ_PAD
N_CHUNKS = B_PER_W // NB
NBUF = 4
LANES = 16
SCALE = math.sqrt(D)

_mesh = plsc.VectorSubcoreMesh(core_axis_name="c", subcore_axis_name="s")


@functools.partial(
    pl.kernel,
    out_type=jax.ShapeDtypeStruct((PART_B * SEQ_PAD, D), jnp.float32),
    mesh=_mesh,
    scratch_types=[
        pltpu.VMEM((ROWS_PER_W,), jnp.int32),
        pltpu.VMEM((NBUF, CHUNK, D), jnp.float32),
        pltpu.VMEM((NBUF, CHUNK, D), jnp.float32),
    ]
    + [pltpu.SemaphoreType.DMA] * (2 * NBUF),
)
def _gather_part(idx_hbm, table_hbm, out_hbm, idx_v, gbuf, sbuf, *sems):
    gsem = sems[:NBUF]
    ssem = sems[NBUF:]
    wid = lax.axis_index("s") * 2 + lax.axis_index("c")
    base = wid * ROWS_PER_W
    pltpu.sync_copy(idx_hbm.at[pl.ds(base, ROWS_PER_W)], idx_v)

    def gather_desc(c, b):
        return pltpu.make_async_copy(
            table_hbm.at[idx_v.at[pl.ds(c * CHUNK, CHUNK)]],
            gbuf.at[b], gsem[b])

    def store_desc(c, b):
        return pltpu.make_async_copy(
            sbuf.at[b], out_hbm.at[pl.ds(base + c * CHUNK, CHUNK)], ssem[b])

    for b in range(NBUF):
        gather_desc(b, b).start()

    for c in range(N_CHUNKS):
        b = c % NBUF
        gather_desc(c, b).wait()

        if c >= NBUF:
            store_desc(c - NBUF, b).wait()

        def scale_row(r, _):
            for j in range(D // LANES):
                sl = pl.ds(j * LANES, LANES)
                sbuf[b, r, sl] = gbuf[b, r, sl] * SCALE
            return 0

        lax.fori_loop(0, CHUNK, scale_row, 0)

        if c + NBUF < N_CHUNKS:
            gather_desc(c + NBUF, b).start()

        store_desc(c, b).start()

    for b in range(NBUF):
        store_desc(N_CHUNKS - NBUF + b, b).wait()


def kernel(input_ids, embed_weight):
    ids = input_ids.astype(jnp.int32)
    idx_pad = jnp.pad(ids, ((0, 0), (0, SEQ_PAD - SEQ)), mode="wrap").reshape(
        BATCH * SEQ_PAD)
    f = jnp.zeros((BATCH, SEQ, D), jnp.float32)
    for k in range(K_PARTS):
        part_idx = lax.slice(
            idx_pad,
            (k * PART_B * SEQ_PAD,),
            ((k + 1) * PART_B * SEQ_PAD,),
        )
        raw2d = _gather_part(part_idx, embed_weight)
        part = raw2d.reshape(PART_B, SEQ_PAD, D)[:, :SEQ, :]
        f = lax.dynamic_update_slice(f, part, (k * PART_B, 0, 0))
    return f
